# baseline (speedup 1.0000x reference)
import functools

import jax
import jax.numpy as jnp
from jax import lax
from jax.experimental import pallas as pl
from jax.experimental.pallas import tpu as pltpu
from jax.experimental.pallas import tpu_sc as plsc

B = 16384
D = 32
K = 20
NC, NS, L = 2, 16, 16
NW = NC * NS
BPW = B // NW
CH = 128
NG = BPW // CH


def _sc_body(ctx_hbm, tgt_hbm, negt_hbm, emb_hbm, out_hbm,
             pos_hbm, neg_hbm,
             idxu, idxv, idxn, u_buf, v_buf, vp_buf, pos_buf, neg_buf, sem):
    wid = lax.axis_index("s") * NC + lax.axis_index("c")
    iota = lax.iota(jnp.int32, L)

    def group(g, carry):
        base = wid * BPW + g * CH
        pltpu.sync_copy(ctx_hbm.at[pl.ds(base, CH)], idxu)
        pltpu.sync_copy(tgt_hbm.at[pl.ds(base, CH)], idxv)
        pltpu.sync_copy(negt_hbm.at[:, pl.ds(base, CH)], idxn)
        cu = pltpu.async_copy(emb_hbm.at[idxu], u_buf, sem)
        cv = pltpu.async_copy(out_hbm.at[idxv], v_buf, sem)
        cps = [pltpu.async_copy(out_hbm.at[idxn.at[j]], vp_buf.at[j], sem)
               for j in range(K)]
        cu.wait()
        cv.wait()
        for c in cps:
            c.wait()

        for s in range(CH // L):
            rb = iota + (s * L)

            def dstep(d, accs):
                cols = jnp.full((L,), d, dtype=jnp.int32)
                u_col = plsc.load_gather(u_buf, [rb, cols])
                v_col = plsc.load_gather(v_buf, [rb, cols])
                new = [accs[0] + u_col * v_col]
                for j in range(K):
                    jj = jnp.full((L,), j, dtype=jnp.int32)
                    c = plsc.load_gather(vp_buf, [jj, rb, cols])
                    new.append(accs[j + 1] + u_col * c)
                return new

            accs = lax.fori_loop(0, D, dstep,
                                 [jnp.zeros((L,), jnp.float32)] * (K + 1))
            pos_buf[pl.ds(s * L, L)] = accs[0]
            for j in range(K):
                neg_buf[j, pl.ds(s * L, L)] = accs[j + 1]

        pltpu.sync_copy(pos_buf, pos_hbm.at[pl.ds(base, CH)])
        pltpu.sync_copy(neg_buf, neg_hbm.at[wid * NG + g])
        return carry

    lax.fori_loop(0, NG, group, 0)


def _sc_sims(context, target, neg_t, emb_table, out_table):
    mesh = plsc.VectorSubcoreMesh(core_axis_name="c", subcore_axis_name="s")
    f = pl.kernel(
        _sc_body,
        out_type=[
            jax.ShapeDtypeStruct((B,), jnp.float32),
            jax.ShapeDtypeStruct((NW * NG, K, CH), jnp.float32),
        ],
        mesh=mesh,
        scratch_types=[
            pltpu.VMEM((CH,), jnp.int32),
            pltpu.VMEM((CH,), jnp.int32),
            pltpu.VMEM((K, CH), jnp.int32),
            pltpu.VMEM((CH, D), jnp.float32),
            pltpu.VMEM((CH, D), jnp.float32),
            pltpu.VMEM((K, CH, D), jnp.float32),
            pltpu.VMEM((CH,), jnp.float32),
            pltpu.VMEM((K, CH), jnp.float32),
            pltpu.SemaphoreType.DMA,
        ],
        compiler_params=pltpu.CompilerParams(needs_layout_passes=False,
                                             use_tc_tiling_on_sc=False),
    )
    return f(context, target, neg_t, emb_table, out_table)


def _tc_loss_body(pos_ref, neg_ref, out_ref):
    p = pos_ref[...]
    n = neg_ref[...]

    def logsig(x):
        return jnp.minimum(x, 0.0) - jnp.log1p(jnp.exp(-jnp.abs(x)))

    total = jnp.sum(logsig(p)) + jnp.sum(logsig(-n))
    out_ref[...] = jnp.reshape(-total / B, (1, 1))


def _tc_loss(pos, neg):
    return pl.pallas_call(
        _tc_loss_body,
        out_shape=jax.ShapeDtypeStruct((1, 1), jnp.float32),
    )(pos.reshape(CH, B // CH), neg.reshape(B * K // CH, CH))


def kernel(context, target, negatives, emb_table, out_table):
    neg_t = negatives.astype(jnp.int32).T
    pos_sims, neg_sims = _sc_sims(context.astype(jnp.int32),
                                  target.astype(jnp.int32),
                                  neg_t, emb_table, out_table)
    loss = _tc_loss(pos_sims, neg_sims.reshape(-1))
    return loss[0, 0]


# speedup vs baseline: 1.0003x; 1.0003x over previous
import functools

import jax
import jax.numpy as jnp
from jax import lax
from jax.experimental import pallas as pl
from jax.experimental.pallas import tpu as pltpu
from jax.experimental.pallas import tpu_sc as plsc

B = 16384
D = 32
K = 20
NC, NS, L = 2, 16, 16
NW = NC * NS
BPW = B // NW
CH = 128
NG = BPW // CH


def _sc_body(ctx_hbm, tgt_hbm, negt_hbm, emb_hbm, out_hbm,
             pos_hbm, neg_hbm,
             idxu, idxv, idxn, u_buf, v_buf, vp_buf, pos_buf, neg_buf, sem):
    wid = lax.axis_index("s") * NC + lax.axis_index("c")
    iota = lax.iota(jnp.int32, L)

    def group(g, carry):
        base = wid * BPW + g * CH
        with jax.named_scope("idx_stage"):
            pltpu.sync_copy(ctx_hbm.at[pl.ds(base, CH)], idxu)
            pltpu.sync_copy(tgt_hbm.at[pl.ds(base, CH)], idxv)
            pltpu.sync_copy(negt_hbm.at[:, pl.ds(base, CH)], idxn)
        with jax.named_scope("gather_fire"):
            cu = pltpu.async_copy(emb_hbm.at[idxu], u_buf, sem)
            cv = pltpu.async_copy(out_hbm.at[idxv], v_buf, sem)
            cps = [pltpu.async_copy(out_hbm.at[idxn.at[j]], vp_buf.at[j], sem)
                   for j in range(K)]
        with jax.named_scope("gather_wait"):
            cu.wait()
            cv.wait()
            for c in cps:
                c.wait()

        for s in range(CH // L):
            rb = iota + (s * L)

            def dstep(d, accs):
                cols = jnp.full((L,), d, dtype=jnp.int32)
                u_col = plsc.load_gather(u_buf, [rb, cols])
                v_col = plsc.load_gather(v_buf, [rb, cols])
                new = [accs[0] + u_col * v_col]
                for j in range(K):
                    jj = jnp.full((L,), j, dtype=jnp.int32)
                    c = plsc.load_gather(vp_buf, [jj, rb, cols])
                    new.append(accs[j + 1] + u_col * c)
                return new

            with jax.named_scope("dots"):
                accs = lax.fori_loop(0, D, dstep,
                                     [jnp.zeros((L,), jnp.float32)] * (K + 1))
            pos_buf[pl.ds(s * L, L)] = accs[0]
            for j in range(K):
                neg_buf[j, pl.ds(s * L, L)] = accs[j + 1]

        with jax.named_scope("writeout"):
            pltpu.sync_copy(pos_buf, pos_hbm.at[pl.ds(base, CH)])
            pltpu.sync_copy(neg_buf, neg_hbm.at[wid * NG + g])
        return carry

    lax.fori_loop(0, NG, group, 0)


def _sc_sims(context, target, neg_t, emb_table, out_table):
    mesh = plsc.VectorSubcoreMesh(core_axis_name="c", subcore_axis_name="s")
    f = pl.kernel(
        _sc_body,
        out_type=[
            jax.ShapeDtypeStruct((B,), jnp.float32),
            jax.ShapeDtypeStruct((NW * NG, K, CH), jnp.float32),
        ],
        mesh=mesh,
        scratch_types=[
            pltpu.VMEM((CH,), jnp.int32),
            pltpu.VMEM((CH,), jnp.int32),
            pltpu.VMEM((K, CH), jnp.int32),
            pltpu.VMEM((CH, D), jnp.float32),
            pltpu.VMEM((CH, D), jnp.float32),
            pltpu.VMEM((K, CH, D), jnp.float32),
            pltpu.VMEM((CH,), jnp.float32),
            pltpu.VMEM((K, CH), jnp.float32),
            pltpu.SemaphoreType.DMA,
        ],
        compiler_params=pltpu.CompilerParams(needs_layout_passes=False,
                                             use_tc_tiling_on_sc=False),
    )
    return f(context, target, neg_t, emb_table, out_table)


def _tc_loss_body(pos_ref, neg_ref, out_ref):
    p = pos_ref[...]
    n = neg_ref[...]

    def logsig(x):
        return jnp.minimum(x, 0.0) - jnp.log1p(jnp.exp(-jnp.abs(x)))

    total = jnp.sum(logsig(p)) + jnp.sum(logsig(-n))
    out_ref[...] = jnp.reshape(-total / B, (1, 1))


def _tc_loss(pos, neg):
    return pl.pallas_call(
        _tc_loss_body,
        out_shape=jax.ShapeDtypeStruct((1, 1), jnp.float32),
    )(pos.reshape(CH, B // CH), neg.reshape(B * K // CH, CH))


def kernel(context, target, negatives, emb_table, out_table):
    neg_t = negatives.astype(jnp.int32).T
    pos_sims, neg_sims = _sc_sims(context.astype(jnp.int32),
                                  target.astype(jnp.int32),
                                  neg_t, emb_table, out_table)
    loss = _tc_loss(pos_sims, neg_sims.reshape(-1))
    return loss[0, 0]
